# baseline (speedup 1.0000x reference)
import jax
import jax.numpy as jnp
from jax import lax
from jax.experimental import pallas as pl
from jax.experimental.pallas import tpu as pltpu

_NUM_ENT = 100000
_NUM_REL = 200
_D = 64
_F = 128
_B = 1024
_NIDX = 3 * _B
_NQ = 2 * _B
_NSPAD = 208

_BJ = 128
_NBJ = _B // _BJ
_BI = 128
_NBI = _B // _BI


def _body(ent_hbm, fidx_ref, w_ref, other_sm_ref, spec_f_ref, mask_q_ref,
          mask_tT_ref, other0_ref, out_ref,
          feats_ref, qs_ref, teT_ref, qb_ref, sem):
    j = pl.program_id(0)

    @pl.when(j == 0)
    def _():
        def issue(k, carry):
            r = fidx_ref[k]
            pltpu.make_async_copy(ent_hbm.at[pl.ds(r, 1)],
                                  feats_ref.at[pl.ds(k, 1)], sem).start()
            return carry

        lax.fori_loop(0, _NIDX, issue, 0, unroll=32)
        pltpu.make_async_copy(ent_hbm.at[pl.ds(0, _NIDX)], feats_ref,
                              sem).wait()

        w = w_ref[...]
        sid = spec_f_ref[...]
        lanes = lax.broadcasted_iota(jnp.int32, (1, _NSPAD), 1)
        onehot = (sid == lanes).astype(jnp.float32)
        specs = jnp.dot(onehot, other_sm_ref[...],
                        preferred_element_type=jnp.float32)
        proj_q = jnp.dot(feats_ref[:_NQ, :], w,
                         preferred_element_type=jnp.float32)
        mq = mask_q_ref[...]
        rows_q = mq * proj_q + (1.0 - mq) * specs
        nrm = jnp.sqrt(jnp.sum(rows_q * rows_q, axis=-1, keepdims=True))
        qn = rows_q / jnp.maximum(nrm, 1e-12)
        qs = qn[:_B] + qn[_B:]
        qs_ref[...] = qs
        projT_t = lax.dot_general(
            w, feats_ref[_NQ:, :], (((0,), (1,)), ((), ())),
            preferred_element_type=jnp.float32)
        mt = mask_tT_ref[...]
        teT = (mt * projT_t
               + (1.0 - mt) * other0_ref[...]).astype(jnp.bfloat16)
        for jj in range(_NBJ):
            teT_ref[jj] = teT[:, jj * _BJ:(jj + 1) * _BJ]
        qsb = qs.astype(jnp.bfloat16)
        for d in range(_D):
            for i in range(_NBI):
                qb_ref[d, i] = jnp.broadcast_to(
                    qsb[i * _BI:(i + 1) * _BI, d:d + 1], (_BI, _BJ))

    tj = teT_ref[j]
    for i in range(_NBI):
        accs = [jnp.zeros((_BI, _BJ), jnp.bfloat16) for _ in range(4)]
        for d in range(_D):
            accs[d % 4] = accs[d % 4] + jnp.abs(qb_ref[d, i] - tj[d:d + 1, :])
        s1 = [accs[0] + accs[1], accs[2] + accs[3]]
        out_ref[pl.ds(i * _BI, _BI), :] = -(s1[0] + s1[1]).astype(jnp.float32)


def kernel(ent_pkl, other_emb, W_proj, batch_input_seqs, target_ent_index):
    seq = batch_input_seqs.astype(jnp.int32)
    t_idx = target_ent_index.astype(jnp.int32)
    all_idx = jnp.concatenate([seq[:, 0], seq[:, 1], t_idx])
    is_ent = (all_idx >= 1) & (all_idx <= _NUM_ENT)
    feat_idx = jnp.where(is_ent, all_idx - 1, 0).astype(jnp.int32)
    q_idx = all_idx[:_NQ]
    spec_idx = jnp.where(q_idx == 0, 0, q_idx - _NUM_ENT)
    spec_idx = jnp.clip(spec_idx, 0, _NUM_REL + 2)

    other_sm = jnp.pad(other_emb, ((0, _NSPAD - (_NUM_REL + 3)), (0, 0)))
    spec_f = spec_idx.astype(jnp.int32)[:, None]

    mask = is_ent.astype(jnp.float32)
    mask_q = mask[:_NQ, None]
    mask_tT = mask[None, _NQ:]
    other0 = other_emb[0][:, None]

    return pl.pallas_call(
        _body,
        grid=(_NBJ,),
        in_specs=[
            pl.BlockSpec(memory_space=pl.ANY),
            pl.BlockSpec(memory_space=pltpu.SMEM),
            pl.BlockSpec((_F, _D), lambda j: (0, 0)),
            pl.BlockSpec((_NSPAD, _D), lambda j: (0, 0)),
            pl.BlockSpec((_NQ, 1), lambda j: (0, 0)),
            pl.BlockSpec((_NQ, 1), lambda j: (0, 0)),
            pl.BlockSpec((1, _B), lambda j: (0, 0)),
            pl.BlockSpec((_D, 1), lambda j: (0, 0)),
        ],
        out_specs=pl.BlockSpec((_B, _BJ), lambda j: (0, j)),
        out_shape=jax.ShapeDtypeStruct((_B, _B), jnp.float32),
        scratch_shapes=[
            pltpu.VMEM((_NIDX, _F), jnp.float32),
            pltpu.VMEM((_B, _D), jnp.float32),
            pltpu.VMEM((_NBJ, _D, _BJ), jnp.bfloat16),
            pltpu.VMEM((_D, _NBI, _BI, _BJ), jnp.bfloat16),
            pltpu.SemaphoreType.DMA,
        ],
    )(ent_pkl, feat_idx, W_proj, other_sm, spec_f, mask_q, mask_tT, other0)


# speedup vs baseline: 1.0009x; 1.0009x over previous
import jax
import jax.numpy as jnp
from jax import lax
from jax.experimental import pallas as pl
from jax.experimental.pallas import tpu as pltpu

_NUM_ENT = 100000
_NUM_REL = 200
_D = 64
_F = 128
_B = 1024
_NIDX = 3 * _B
_NQ = 2 * _B
_NSPAD = 208

_BJ = 128
_NBJ = _B // _BJ
_BI = 128
_NBI = _B // _BI


def _body(ent_hbm, fidx_ref, w_ref, other_sm_ref, spec_f_ref, mask_q_ref,
          mask_tT_ref, other0_ref, out_ref,
          feats_ref, qs_ref, teT_ref, qb_ref, sem):
    j = pl.program_id(0)

    @pl.when(j == 0)
    def _():
        def issue(k, carry):
            r = fidx_ref[k]
            pltpu.make_async_copy(ent_hbm.at[pl.ds(r, 1)],
                                  feats_ref.at[pl.ds(k, 1)], sem).start()
            return carry

        lax.fori_loop(0, _NIDX, issue, 0, unroll=8)
        pltpu.make_async_copy(ent_hbm.at[pl.ds(0, _NIDX)], feats_ref,
                              sem).wait()

        w = w_ref[...]
        sid = spec_f_ref[...]
        lanes = lax.broadcasted_iota(jnp.int32, (1, _NSPAD), 1)
        onehot = (sid == lanes).astype(jnp.float32)
        specs = jnp.dot(onehot, other_sm_ref[...],
                        preferred_element_type=jnp.float32)
        proj_q = jnp.dot(feats_ref[:_NQ, :], w,
                         preferred_element_type=jnp.float32)
        mq = mask_q_ref[...]
        rows_q = mq * proj_q + (1.0 - mq) * specs
        nrm = jnp.sqrt(jnp.sum(rows_q * rows_q, axis=-1, keepdims=True))
        qn = rows_q / jnp.maximum(nrm, 1e-12)
        qs = qn[:_B] + qn[_B:]
        qs_ref[...] = qs
        projT_t = lax.dot_general(
            w, feats_ref[_NQ:, :], (((0,), (1,)), ((), ())),
            preferred_element_type=jnp.float32)
        mt = mask_tT_ref[...]
        teT = (mt * projT_t
               + (1.0 - mt) * other0_ref[...]).astype(jnp.bfloat16)
        for jj in range(_NBJ):
            teT_ref[jj] = teT[:, jj * _BJ:(jj + 1) * _BJ]
        qsb = qs.astype(jnp.bfloat16)
        for d in range(_D):
            for i in range(_NBI):
                qb_ref[d, i] = jnp.broadcast_to(
                    qsb[i * _BI:(i + 1) * _BI, d:d + 1], (_BI, _BJ))

    tj = teT_ref[j]
    for i in range(_NBI):
        accs = [jnp.zeros((_BI, _BJ), jnp.bfloat16) for _ in range(4)]
        for d in range(_D):
            accs[d % 4] = accs[d % 4] + jnp.abs(qb_ref[d, i] - tj[d:d + 1, :])
        s1 = [accs[0] + accs[1], accs[2] + accs[3]]
        out_ref[pl.ds(i * _BI, _BI), :] = -(s1[0] + s1[1]).astype(jnp.float32)


def kernel(ent_pkl, other_emb, W_proj, batch_input_seqs, target_ent_index):
    seq = batch_input_seqs.astype(jnp.int32)
    t_idx = target_ent_index.astype(jnp.int32)
    all_idx = jnp.concatenate([seq[:, 0], seq[:, 1], t_idx])
    is_ent = (all_idx >= 1) & (all_idx <= _NUM_ENT)
    feat_idx = jnp.where(is_ent, all_idx - 1, 0).astype(jnp.int32)
    q_idx = all_idx[:_NQ]
    spec_idx = jnp.where(q_idx == 0, 0, q_idx - _NUM_ENT)
    spec_idx = jnp.clip(spec_idx, 0, _NUM_REL + 2)

    other_sm = jnp.pad(other_emb, ((0, _NSPAD - (_NUM_REL + 3)), (0, 0)))
    spec_f = spec_idx.astype(jnp.int32)[:, None]

    mask = is_ent.astype(jnp.float32)
    mask_q = mask[:_NQ, None]
    mask_tT = mask[None, _NQ:]
    other0 = other_emb[0][:, None]

    return pl.pallas_call(
        _body,
        grid=(_NBJ,),
        in_specs=[
            pl.BlockSpec(memory_space=pl.ANY),
            pl.BlockSpec(memory_space=pltpu.SMEM),
            pl.BlockSpec((_F, _D), lambda j: (0, 0)),
            pl.BlockSpec((_NSPAD, _D), lambda j: (0, 0)),
            pl.BlockSpec((_NQ, 1), lambda j: (0, 0)),
            pl.BlockSpec((_NQ, 1), lambda j: (0, 0)),
            pl.BlockSpec((1, _B), lambda j: (0, 0)),
            pl.BlockSpec((_D, 1), lambda j: (0, 0)),
        ],
        out_specs=pl.BlockSpec((_B, _BJ), lambda j: (0, j)),
        out_shape=jax.ShapeDtypeStruct((_B, _B), jnp.float32),
        scratch_shapes=[
            pltpu.VMEM((_NIDX, _F), jnp.float32),
            pltpu.VMEM((_B, _D), jnp.float32),
            pltpu.VMEM((_NBJ, _D, _BJ), jnp.bfloat16),
            pltpu.VMEM((_D, _NBI, _BI, _BJ), jnp.bfloat16),
            pltpu.SemaphoreType.DMA,
        ],
    )(ent_pkl, feat_idx, W_proj, other_sm, spec_f, mask_q, mask_tT, other0)
